# baseline (speedup 1.0000x reference)
import functools

import jax
import jax.numpy as jnp
from jax import lax
from jax.experimental import pallas as pl
from jax.experimental.pallas import tpu as pltpu
from jax.experimental.pallas import tpu_sc as plsc

_MIN_VAL = 0
_MAX_VAL = 999999
_D = 64
_B = 16384

_info = plsc.get_sparse_core_info()
_NC, _NS, _L = _info.num_cores, _info.num_subcores, _info.num_lanes
_NW = _NC * _NS
_BPW = _B // _NW

_mesh = plsc.VectorSubcoreMesh(core_axis_name="c", subcore_axis_name="s")


@functools.partial(
    pl.kernel,
    mesh=_mesh,
    out_type=jax.ShapeDtypeStruct((_B, _D), jnp.float32),
    compiler_params=pltpu.CompilerParams(use_tc_tiling_on_sc=False),
    scratch_types=[
        pltpu.VMEM((_BPW,), jnp.int32),
        pltpu.VMEM((_BPW, _D), jnp.float32),
        pltpu.SemaphoreType.DMA,
    ],
)
def _gather_rows(ints_hbm, table_hbm, out_hbm, idx_v, rows_v, sem):
    wid = lax.axis_index("s") * _NC + lax.axis_index("c")
    base = wid * _BPW
    pltpu.sync_copy(ints_hbm.at[pl.ds(base, _BPW)], idx_v)
    for i in range(_BPW // _L):
        sl = pl.ds(i * _L, _L)
        idx_v[sl] = jnp.clip(idx_v[sl], _MIN_VAL, _MAX_VAL)
    pltpu.async_copy(table_hbm.at[idx_v], rows_v, sem).wait()
    pltpu.sync_copy(rows_v, out_hbm.at[pl.ds(base, _BPW)])


def kernel(ints, table):
    rows = _gather_rows(ints.astype(jnp.int32), table)
    int_embeds = rows.reshape(_B, 1, _D)
    mask = jnp.ones((_B, 1), dtype=jnp.float32)
    return (int_embeds, mask)


# speedup vs baseline: 1.7039x; 1.7039x over previous
import functools

import jax
import jax.numpy as jnp
from jax import lax
from jax.experimental import pallas as pl
from jax.experimental.pallas import tpu as pltpu
from jax.experimental.pallas import tpu_sc as plsc

_MIN_VAL = 0
_MAX_VAL = 999999
_D = 64
_B = 16384

_info = plsc.get_sparse_core_info()
_NC, _NS, _L = _info.num_cores, _info.num_subcores, _info.num_lanes
_NW = _NC * _NS
_BPW = _B // _NW

_mesh = plsc.VectorSubcoreMesh(core_axis_name="c", subcore_axis_name="s")


@functools.partial(
    pl.kernel,
    mesh=_mesh,
    out_type=jax.ShapeDtypeStruct((_B, _D), jnp.float32),
    compiler_params=pltpu.CompilerParams(needs_layout_passes=False),
    scratch_types=[
        pltpu.VMEM((_BPW,), jnp.int32),
        pltpu.VMEM((_BPW, _D), jnp.float32),
        pltpu.SemaphoreType.DMA,
    ],
)
def _gather_rows(ints_hbm, table_hbm, out_hbm, idx_v, rows_v, sem):
    wid = lax.axis_index("s") * _NC + lax.axis_index("c")
    base = wid * _BPW
    pltpu.sync_copy(ints_hbm.at[pl.ds(base, _BPW)], idx_v)
    copies = []
    for i in range(_BPW // _L):
        vec = jnp.clip(idx_v[pl.ds(i * _L, _L)], _MIN_VAL, _MAX_VAL)
        for l in range(_L):
            r = i * _L + l
            c = pltpu.make_async_copy(
                table_hbm.at[pl.ds(vec[l], 1)], rows_v.at[pl.ds(r, 1)], sem
            )
            c.start()
            copies.append(c)
    for c in copies:
        c.wait()
    pltpu.sync_copy(rows_v, out_hbm.at[pl.ds(base, _BPW)])


def kernel(ints, table):
    rows = _gather_rows(ints.astype(jnp.int32), table)
    int_embeds = rows.reshape(_B, 1, _D)
    mask = jnp.ones((_B, 1), dtype=jnp.float32)
    return (int_embeds, mask)
